# baseline (speedup 1.0000x reference)
import functools

import jax
import jax.numpy as jnp
from jax import lax
from jax.experimental import pallas as pl
from jax.experimental.pallas import tpu as pltpu
from jax.experimental.pallas import tpu_sc as plsc

N_NODES = 10000
N_EDGES = 320000
D = 128

NC = 2
NS = 16
NW = NC * NS

NP = 10240
EPT = NP
CH = 128
NCHUNK = EPT // CH
IB = 16
NREFILL = NCHUNK // IB
ROWS_PER_TILE = NP // NS
WB = ROWS_PER_TILE // CH

_mesh = plsc.VectorSubcoreMesh(core_axis_name="c", subcore_axis_name="s",
                               num_cores=NC, num_subcores=NS)


@functools.partial(
    pl.kernel,
    out_type=jax.ShapeDtypeStruct((NC, NP, D), jnp.float32),
    mesh=_mesh,
    scratch_types=[
        pltpu.VMEM((IB, CH), jnp.int32),
        pltpu.VMEM((IB, CH), jnp.int32),
        pltpu.VMEM((CH, D), jnp.float32),
        pltpu.VMEM((CH, D), jnp.float32),
        pltpu.VMEM_SHARED((NP, D), jnp.float32),
        pltpu.SemaphoreType.DMA,
        pltpu.SemaphoreType.DMA,
    ],
)
def _agg_kernel(hs_hbm, src_hbm, dst_hbm, zeros_hbm, part_hbm,
                srcv, dstv, b0, b1, acc, sem0, sem1):
    cid = lax.axis_index("c")
    sid = lax.axis_index("s")
    wid = cid * NS + sid

    pltpu.sync_copy(zeros_hbm, b0)
    row0 = sid * ROWS_PER_TILE
    for kk in range(WB):
        pltpu.sync_copy(b0, acc.at[pl.ds(row0 + kk * CH, CH)])
    plsc.subcore_barrier()

    def outer(r, _):
        pltpu.sync_copy(src_hbm.at[wid, pl.ds(r * IB, IB)], srcv)
        pltpu.sync_copy(dst_hbm.at[wid, pl.ds(r * IB, IB)], dstv)

        def body(i, _):
            j0 = 2 * i
            j1 = j0 + 1
            cp0 = pltpu.async_copy(hs_hbm.at[srcv.at[j0]], b0, sem0)

            @pl.when(i > 0)
            def _():
                pltpu.sync_copy(b1, acc.at[dstv.at[j0 - 1]], add=True)

            cp0.wait()
            cp1 = pltpu.async_copy(hs_hbm.at[srcv.at[j1]], b1, sem1)
            pltpu.sync_copy(b0, acc.at[dstv.at[j0]], add=True)
            cp1.wait()
            return 0
        lax.fori_loop(0, IB // 2, body, 0)
        pltpu.sync_copy(b1, acc.at[dstv.at[IB - 1]], add=True)
        return 0
    lax.fori_loop(0, NREFILL, outer, 0)
    plsc.subcore_barrier()

    def wb(kk, _):
        r = row0 + kk * CH
        pltpu.sync_copy(acc.at[pl.ds(r, CH)], b0)
        pltpu.sync_copy(b0, part_hbm.at[cid, pl.ds(r, CH)])
        return 0
    lax.fori_loop(0, WB, wb, 0)


BM = 1280
GRID = NP // BM


def _prep_body(degp_ref, x_ref, w_ref, hs_ref, dis_ref):
    deg = degp_ref[0, :, 0:1] + degp_ref[1, :, 0:1] + 1.0
    dis = jnp.broadcast_to(lax.rsqrt(deg), (BM, D))
    hs_ref[...] = jnp.dot(x_ref[...], w_ref[...],
                          preferred_element_type=jnp.float32) * dis
    dis_ref[...] = dis


_prep_call = pl.pallas_call(
    _prep_body,
    grid=(GRID,),
    in_specs=[
        pl.BlockSpec((NC, BM, D), lambda i: (0, i, 0)),
        pl.BlockSpec((BM, D), lambda i: (i, 0)),
        pl.BlockSpec((D, D), lambda i: (0, 0)),
    ],
    out_specs=[
        pl.BlockSpec((BM, D), lambda i: (i, 0)),
        pl.BlockSpec((BM, D), lambda i: (i, 0)),
    ],
    out_shape=[
        jax.ShapeDtypeStruct((NP, D), jnp.float32),
        jax.ShapeDtypeStruct((NP, D), jnp.float32),
    ],
)


def _mid_body(part_ref, hs_ref, dis_ref, b_ref, w_ref, out_ref):
    agg = part_ref[0] + part_ref[1] + hs_ref[...]
    o = dis_ref[...] * agg + b_ref[...]
    r = jnp.maximum(o, 0.0)
    out_ref[...] = jnp.dot(r, w_ref[...],
                           preferred_element_type=jnp.float32) * dis_ref[...]


_mid_call = pl.pallas_call(
    _mid_body,
    grid=(GRID,),
    in_specs=[
        pl.BlockSpec((NC, BM, D), lambda i: (0, i, 0)),
        pl.BlockSpec((BM, D), lambda i: (i, 0)),
        pl.BlockSpec((BM, D), lambda i: (i, 0)),
        pl.BlockSpec((1, D), lambda i: (0, 0)),
        pl.BlockSpec((D, D), lambda i: (0, 0)),
    ],
    out_specs=pl.BlockSpec((BM, D), lambda i: (i, 0)),
    out_shape=jax.ShapeDtypeStruct((NP, D), jnp.float32),
)


def _final_body(part_ref, hs_ref, dis_ref, b_ref, out_ref):
    agg = part_ref[0] + part_ref[1] + hs_ref[...]
    out_ref[...] = dis_ref[...] * agg + b_ref[...]


_final_call = pl.pallas_call(
    _final_body,
    grid=(GRID,),
    in_specs=[
        pl.BlockSpec((NC, BM, D), lambda i: (0, i, 0)),
        pl.BlockSpec((BM, D), lambda i: (i, 0)),
        pl.BlockSpec((BM, D), lambda i: (i, 0)),
        pl.BlockSpec((1, D), lambda i: (0, 0)),
    ],
    out_specs=pl.BlockSpec((BM, D), lambda i: (i, 0)),
    out_shape=jax.ShapeDtypeStruct((NP, D), jnp.float32),
)


def kernel(x, edge_index, W1, b1, W2, b2, W3, b3):
    src = edge_index[0].astype(jnp.int32)
    dst = edge_index[1].astype(jnp.int32)

    real_per_tile = N_EDGES // NW
    pad_per_tile = EPT - real_per_tile
    src_t = jnp.concatenate(
        [src.reshape(NW, real_per_tile),
         jnp.zeros((NW, pad_per_tile), jnp.int32)], axis=1
    ).reshape(NW, NCHUNK, CH)
    trash = N_NODES + jnp.arange(pad_per_tile, dtype=jnp.int32)
    dst_t = jnp.concatenate(
        [dst.reshape(NW, real_per_tile),
         jnp.broadcast_to(trash, (NW, pad_per_tile))], axis=1
    ).reshape(NW, NCHUNK, CH)

    ones8 = jnp.ones((8, D), jnp.float32)
    srcz_t = jnp.zeros_like(src_t)
    zeros_hbm = jnp.zeros((CH, D), jnp.float32)
    xp = jnp.zeros((NP, D), jnp.float32).at[:N_NODES].set(x)
    b1r = b1.reshape(1, D)
    b2r = b2.reshape(1, D)
    b3r = b3.reshape(1, D)

    degp = _agg_kernel(ones8, srcz_t, dst_t, zeros_hbm)
    hs1, dis = _prep_call(degp, xp, W1)
    p1 = _agg_kernel(hs1, src_t, dst_t, zeros_hbm)
    hs2 = _mid_call(p1, hs1, dis, b1r, W2)
    p2 = _agg_kernel(hs2, src_t, dst_t, zeros_hbm)
    hs3 = _mid_call(p2, hs2, dis, b2r, W3)
    p3 = _agg_kernel(hs3, src_t, dst_t, zeros_hbm)
    z = _final_call(p3, hs3, dis, b3r)
    return z[:N_NODES]


# speedup vs baseline: 7.7959x; 7.7959x over previous
import functools

import jax
import jax.numpy as jnp
from jax import lax
from jax.experimental import pallas as pl
from jax.experimental.pallas import tpu as pltpu
from jax.experimental.pallas import tpu_sc as plsc

N_NODES = 10000
N_EDGES = 320000
D = 128

NC = 2
NS = 16
NW = NC * NS

NP = 10240
EPT = NP
CH = 128
NCHUNK = EPT // CH
IB = 16
NREFILL = NCHUNK // IB
ROWS_PER_TILE = NP // NS
WB = ROWS_PER_TILE // CH

_mesh = plsc.VectorSubcoreMesh(core_axis_name="c", subcore_axis_name="s",
                               num_cores=NC, num_subcores=NS)


@functools.partial(
    pl.kernel,
    out_type=jax.ShapeDtypeStruct((NC, NP, D), jnp.float32),
    mesh=_mesh,
    scratch_types=[
        pltpu.VMEM((IB, CH), jnp.int32),
        pltpu.VMEM((IB, CH), jnp.int32),
        pltpu.VMEM((CH, D), jnp.float32),
        pltpu.VMEM((CH, D), jnp.float32),
        pltpu.VMEM_SHARED((NP, D), jnp.float32),
        pltpu.SemaphoreType.DMA,
        pltpu.SemaphoreType.DMA,
    ],
)
def _agg_kernel(hs_hbm, src_hbm, dst_hbm, zeros_hbm, part_hbm,
                srcv, dstv, b0, b1, acc, sem0, sem1):
    cid = lax.axis_index("c")
    sid = lax.axis_index("s")
    wid = cid * NS + sid

    pltpu.sync_copy(zeros_hbm, b0)
    row0 = sid * ROWS_PER_TILE
    for kk in range(WB):
        pltpu.sync_copy(b0, acc.at[pl.ds(row0 + kk * CH, CH)])
    plsc.subcore_barrier()

    def outer(r, _):
        pltpu.sync_copy(src_hbm.at[wid, pl.ds(r * IB, IB)], srcv)
        pltpu.sync_copy(dst_hbm.at[wid, pl.ds(r * IB, IB)], dstv)

        def body(i, _):
            j0 = 2 * i
            j1 = j0 + 1
            cp0 = pltpu.async_copy(hs_hbm.at[srcv.at[j0]], b0, sem0)

            @pl.when(i > 0)
            def _():
                pltpu.sync_copy(b1, acc.at[dstv.at[j0 - 1]], add=True)

            cp0.wait()
            cp1 = pltpu.async_copy(hs_hbm.at[srcv.at[j1]], b1, sem1)
            pltpu.sync_copy(b0, acc.at[dstv.at[j0]], add=True)
            cp1.wait()
            return 0
        lax.fori_loop(0, IB // 2, body, 0)
        pltpu.sync_copy(b1, acc.at[dstv.at[IB - 1]], add=True)
        return 0
    lax.fori_loop(0, NREFILL, outer, 0)
    plsc.subcore_barrier()

    def wb(kk, _):
        r = row0 + kk * CH
        pltpu.sync_copy(acc.at[pl.ds(r, CH)], b0)
        pltpu.sync_copy(b0, part_hbm.at[cid, pl.ds(r, CH)])
        return 0
    lax.fori_loop(0, WB, wb, 0)


BM = 1280
GRID = NP // BM


def _prep_body(degp_ref, x_ref, w_ref, hs_ref, dis_ref):
    deg = degp_ref[0, :, 0:1] + degp_ref[1, :, 0:1] + 1.0
    dis = jnp.broadcast_to(lax.rsqrt(deg), (BM, D))
    hs_ref[...] = jnp.dot(x_ref[...], w_ref[...],
                          preferred_element_type=jnp.float32) * dis
    dis_ref[...] = dis


_prep_call = pl.pallas_call(
    _prep_body,
    grid=(GRID,),
    in_specs=[
        pl.BlockSpec((NC, BM, D), lambda i: (0, i, 0)),
        pl.BlockSpec((BM, D), lambda i: (i, 0)),
        pl.BlockSpec((D, D), lambda i: (0, 0)),
    ],
    out_specs=[
        pl.BlockSpec((BM, D), lambda i: (i, 0)),
        pl.BlockSpec((BM, D), lambda i: (i, 0)),
    ],
    out_shape=[
        jax.ShapeDtypeStruct((NP, D), jnp.float32),
        jax.ShapeDtypeStruct((NP, D), jnp.float32),
    ],
)


def _mid_body(part_ref, hs_ref, dis_ref, b_ref, w_ref, out_ref):
    agg = part_ref[0] + part_ref[1] + hs_ref[...]
    o = dis_ref[...] * agg + b_ref[...]
    r = jnp.maximum(o, 0.0)
    out_ref[...] = jnp.dot(r, w_ref[...],
                           preferred_element_type=jnp.float32) * dis_ref[...]


_mid_call = pl.pallas_call(
    _mid_body,
    grid=(GRID,),
    in_specs=[
        pl.BlockSpec((NC, BM, D), lambda i: (0, i, 0)),
        pl.BlockSpec((BM, D), lambda i: (i, 0)),
        pl.BlockSpec((BM, D), lambda i: (i, 0)),
        pl.BlockSpec((1, D), lambda i: (0, 0)),
        pl.BlockSpec((D, D), lambda i: (0, 0)),
    ],
    out_specs=pl.BlockSpec((BM, D), lambda i: (i, 0)),
    out_shape=jax.ShapeDtypeStruct((NP, D), jnp.float32),
)


def _final_body(part_ref, hs_ref, dis_ref, b_ref, out_ref):
    agg = part_ref[0] + part_ref[1] + hs_ref[...]
    out_ref[...] = dis_ref[...] * agg + b_ref[...]


_final_call = pl.pallas_call(
    _final_body,
    grid=(GRID,),
    in_specs=[
        pl.BlockSpec((NC, BM, D), lambda i: (0, i, 0)),
        pl.BlockSpec((BM, D), lambda i: (i, 0)),
        pl.BlockSpec((BM, D), lambda i: (i, 0)),
        pl.BlockSpec((1, D), lambda i: (0, 0)),
    ],
    out_specs=pl.BlockSpec((BM, D), lambda i: (i, 0)),
    out_shape=jax.ShapeDtypeStruct((NP, D), jnp.float32),
)


def kernel(x, edge_index, W1, b1, W2, b2, W3, b3):
    src = edge_index[0].astype(jnp.int32)
    dst = edge_index[1].astype(jnp.int32)

    real_per_tile = N_EDGES // NW
    pad_per_tile = EPT - real_per_tile
    src_t = jnp.concatenate(
        [src.reshape(NW, real_per_tile),
         jnp.zeros((NW, pad_per_tile), jnp.int32)], axis=1
    ).reshape(NW, NCHUNK, CH)
    trash = N_NODES + jnp.arange(pad_per_tile, dtype=jnp.int32)
    dst_t = jnp.concatenate(
        [dst.reshape(NW, real_per_tile),
         jnp.broadcast_to(trash, (NW, pad_per_tile))], axis=1
    ).reshape(NW, NCHUNK, CH)

    onesm = jnp.ones((NP, D), jnp.float32)
    zeros_hbm = jnp.zeros((CH, D), jnp.float32)
    xp = jnp.zeros((NP, D), jnp.float32).at[:N_NODES].set(x)
    b1r = b1.reshape(1, D)
    b2r = b2.reshape(1, D)
    b3r = b3.reshape(1, D)

    degp = _agg_kernel(onesm, src_t, dst_t, zeros_hbm)
    hs1, dis = _prep_call(degp, xp, W1)
    p1 = _agg_kernel(hs1, src_t, dst_t, zeros_hbm)
    hs2 = _mid_call(p1, hs1, dis, b1r, W2)
    p2 = _agg_kernel(hs2, src_t, dst_t, zeros_hbm)
    hs3 = _mid_call(p2, hs2, dis, b2r, W3)
    p3 = _agg_kernel(hs3, src_t, dst_t, zeros_hbm)
    z = _final_call(p3, hs3, dis, b3r)
    return z[:N_NODES]


# speedup vs baseline: 9.6815x; 1.2419x over previous
import functools

import jax
import jax.numpy as jnp
from jax import lax
from jax.experimental import pallas as pl
from jax.experimental.pallas import tpu as pltpu
from jax.experimental.pallas import tpu_sc as plsc

N_NODES = 10000
N_EDGES = 320000
D = 128

NC = 2
NS = 16
NW = NC * NS

NP = 10240
EPT = NP
CH = 128
NCHUNK = EPT // CH
IB = 16
NREFILL = NCHUNK // IB
ROWS_PER_TILE = NP // NS
WB = ROWS_PER_TILE // CH

_mesh = plsc.VectorSubcoreMesh(core_axis_name="c", subcore_axis_name="s",
                               num_cores=NC, num_subcores=NS)


@functools.partial(
    pl.kernel,
    out_type=jax.ShapeDtypeStruct((NC, NP, D), jnp.float32),
    mesh=_mesh,
    scratch_types=[
        pltpu.VMEM((IB, CH), jnp.int32),
        pltpu.VMEM((CH, D), jnp.float32),
        pltpu.VMEM((CH, D), jnp.float32),
        pltpu.VMEM_SHARED((NP, D), jnp.float32),
    ],
)
def _deg_kernel(dst_hbm, ones_hbm, zeros_hbm, degp_hbm, dstv, onesv, wbuf, acc):
    cid = lax.axis_index("c")
    sid = lax.axis_index("s")
    wid = cid * NS + sid
    pltpu.sync_copy(ones_hbm, onesv)
    pltpu.sync_copy(zeros_hbm, wbuf)
    row0 = sid * ROWS_PER_TILE
    for kk in range(WB):
        pltpu.sync_copy(wbuf, acc.at[pl.ds(row0 + kk * CH, CH)])
    plsc.subcore_barrier()

    def outer(r, _):
        pltpu.sync_copy(dst_hbm.at[wid, pl.ds(r * IB, IB)], dstv)

        def body(j, _):
            pltpu.sync_copy(onesv, acc.at[dstv.at[j]], add=True)
            return 0
        lax.fori_loop(0, IB, body, 0)
        return 0
    lax.fori_loop(0, NREFILL, outer, 0)
    plsc.subcore_barrier()

    def wb(kk, _):
        r = row0 + kk * CH
        pltpu.sync_copy(acc.at[pl.ds(r, CH)], wbuf)
        pltpu.sync_copy(wbuf, degp_hbm.at[cid, pl.ds(r, CH)])
        return 0
    lax.fori_loop(0, WB, wb, 0)


@functools.partial(
    pl.kernel,
    out_type=jax.ShapeDtypeStruct((NC, NP, D), jnp.float32),
    mesh=_mesh,
    scratch_types=[
        pltpu.VMEM((IB, CH), jnp.int32),
        pltpu.VMEM((IB, CH), jnp.int32),
        pltpu.VMEM((CH, D), jnp.float32),
        pltpu.VMEM((CH, D), jnp.float32),
        pltpu.VMEM_SHARED((NP, D), jnp.float32),
        pltpu.SemaphoreType.DMA,
        pltpu.SemaphoreType.DMA,
    ],
)
def _agg_kernel(hs_hbm, src_hbm, dst_hbm, zeros_hbm, part_hbm,
                srcv, dstv, b0, b1, acc, sem0, sem1):
    cid = lax.axis_index("c")
    sid = lax.axis_index("s")
    wid = cid * NS + sid

    pltpu.sync_copy(zeros_hbm, b0)
    row0 = sid * ROWS_PER_TILE
    for kk in range(WB):
        pltpu.sync_copy(b0, acc.at[pl.ds(row0 + kk * CH, CH)])
    plsc.subcore_barrier()

    def outer(r, _):
        pltpu.sync_copy(src_hbm.at[wid, pl.ds(r * IB, IB)], srcv)
        pltpu.sync_copy(dst_hbm.at[wid, pl.ds(r * IB, IB)], dstv)

        def body(i, _):
            j0 = 2 * i
            j1 = j0 + 1
            cp0 = pltpu.async_copy(hs_hbm.at[srcv.at[j0]], b0, sem0)

            @pl.when(i > 0)
            def _():
                pltpu.sync_copy(b1, acc.at[dstv.at[j0 - 1]], add=True)

            cp0.wait()
            cp1 = pltpu.async_copy(hs_hbm.at[srcv.at[j1]], b1, sem1)
            pltpu.sync_copy(b0, acc.at[dstv.at[j0]], add=True)
            cp1.wait()
            return 0
        lax.fori_loop(0, IB // 2, body, 0)
        pltpu.sync_copy(b1, acc.at[dstv.at[IB - 1]], add=True)
        return 0
    lax.fori_loop(0, NREFILL, outer, 0)
    plsc.subcore_barrier()

    def wb(kk, _):
        r = row0 + kk * CH
        pltpu.sync_copy(acc.at[pl.ds(r, CH)], b0)
        pltpu.sync_copy(b0, part_hbm.at[cid, pl.ds(r, CH)])
        return 0
    lax.fori_loop(0, WB, wb, 0)


BM = 1280
GRID = NP // BM


def _prep_body(degp_ref, x_ref, w_ref, hs_ref, dis_ref):
    deg = degp_ref[0, :, 0:1] + degp_ref[1, :, 0:1] + 1.0
    dis = jnp.broadcast_to(lax.rsqrt(deg), (BM, D))
    hs_ref[...] = jnp.dot(x_ref[...], w_ref[...],
                          preferred_element_type=jnp.float32) * dis
    dis_ref[...] = dis


_prep_call = pl.pallas_call(
    _prep_body,
    grid=(GRID,),
    in_specs=[
        pl.BlockSpec((NC, BM, D), lambda i: (0, i, 0)),
        pl.BlockSpec((BM, D), lambda i: (i, 0)),
        pl.BlockSpec((D, D), lambda i: (0, 0)),
    ],
    out_specs=[
        pl.BlockSpec((BM, D), lambda i: (i, 0)),
        pl.BlockSpec((BM, D), lambda i: (i, 0)),
    ],
    out_shape=[
        jax.ShapeDtypeStruct((NP, D), jnp.float32),
        jax.ShapeDtypeStruct((NP, D), jnp.float32),
    ],
)


def _mid_body(part_ref, hs_ref, dis_ref, b_ref, w_ref, out_ref):
    agg = part_ref[0] + part_ref[1] + hs_ref[...]
    o = dis_ref[...] * agg + b_ref[...]
    r = jnp.maximum(o, 0.0)
    out_ref[...] = jnp.dot(r, w_ref[...],
                           preferred_element_type=jnp.float32) * dis_ref[...]


_mid_call = pl.pallas_call(
    _mid_body,
    grid=(GRID,),
    in_specs=[
        pl.BlockSpec((NC, BM, D), lambda i: (0, i, 0)),
        pl.BlockSpec((BM, D), lambda i: (i, 0)),
        pl.BlockSpec((BM, D), lambda i: (i, 0)),
        pl.BlockSpec((1, D), lambda i: (0, 0)),
        pl.BlockSpec((D, D), lambda i: (0, 0)),
    ],
    out_specs=pl.BlockSpec((BM, D), lambda i: (i, 0)),
    out_shape=jax.ShapeDtypeStruct((NP, D), jnp.float32),
)


def _final_body(part_ref, hs_ref, dis_ref, b_ref, out_ref):
    agg = part_ref[0] + part_ref[1] + hs_ref[...]
    out_ref[...] = dis_ref[...] * agg + b_ref[...]


_final_call = pl.pallas_call(
    _final_body,
    grid=(GRID,),
    in_specs=[
        pl.BlockSpec((NC, BM, D), lambda i: (0, i, 0)),
        pl.BlockSpec((BM, D), lambda i: (i, 0)),
        pl.BlockSpec((BM, D), lambda i: (i, 0)),
        pl.BlockSpec((1, D), lambda i: (0, 0)),
    ],
    out_specs=pl.BlockSpec((BM, D), lambda i: (i, 0)),
    out_shape=jax.ShapeDtypeStruct((NP, D), jnp.float32),
)


def kernel(x, edge_index, W1, b1, W2, b2, W3, b3):
    src = edge_index[0].astype(jnp.int32)
    dst = edge_index[1].astype(jnp.int32)

    real_per_tile = N_EDGES // NW
    pad_per_tile = EPT - real_per_tile
    src_t = jnp.concatenate(
        [src.reshape(NW, real_per_tile),
         jnp.zeros((NW, pad_per_tile), jnp.int32)], axis=1
    ).reshape(NW, NCHUNK, CH)
    trash = N_NODES + jnp.arange(pad_per_tile, dtype=jnp.int32)
    dst_t = jnp.concatenate(
        [dst.reshape(NW, real_per_tile),
         jnp.broadcast_to(trash, (NW, pad_per_tile))], axis=1
    ).reshape(NW, NCHUNK, CH)

    ones_hbm = jnp.ones((CH, D), jnp.float32)
    zeros_hbm = jnp.zeros((CH, D), jnp.float32)
    xp = jnp.zeros((NP, D), jnp.float32).at[:N_NODES].set(x)
    b1r = b1.reshape(1, D)
    b2r = b2.reshape(1, D)
    b3r = b3.reshape(1, D)

    degp = _deg_kernel(dst_t, ones_hbm, zeros_hbm)
    hs1, dis = _prep_call(degp, xp, W1)
    p1 = _agg_kernel(hs1, src_t, dst_t, zeros_hbm)
    hs2 = _mid_call(p1, hs1, dis, b1r, W2)
    p2 = _agg_kernel(hs2, src_t, dst_t, zeros_hbm)
    hs3 = _mid_call(p2, hs2, dis, b2r, W3)
    p3 = _agg_kernel(hs3, src_t, dst_t, zeros_hbm)
    z = _final_call(p3, hs3, dis, b3r)
    return z[:N_NODES]


# speedup vs baseline: 10.1828x; 1.0518x over previous
import functools

import jax
import jax.numpy as jnp
from jax import lax
from jax.experimental import pallas as pl
from jax.experimental.pallas import tpu as pltpu
from jax.experimental.pallas import tpu_sc as plsc

N_NODES = 10000
N_EDGES = 320000
D = 128

NC = 2
NS = 16
NW = NC * NS

NP = 10240
EPT = NP
CH = 128
NCHUNK = EPT // CH
IB = 16
NREFILL = NCHUNK // IB
ROWS_PER_TILE = NP // NS
WB = ROWS_PER_TILE // CH

ACH = 64
ANCHUNK = EPT // ACH
AIB = 32
ANREFILL = ANCHUNK // AIB
NBUF = 4
AWB = ROWS_PER_TILE // ACH

_mesh = plsc.VectorSubcoreMesh(core_axis_name="c", subcore_axis_name="s",
                               num_cores=NC, num_subcores=NS)


@functools.partial(
    pl.kernel,
    out_type=jax.ShapeDtypeStruct((NC, NP, D), jnp.float32),
    mesh=_mesh,
    scratch_types=[
        pltpu.VMEM((IB, CH), jnp.int32),
        pltpu.VMEM((CH, D), jnp.float32),
        pltpu.VMEM((CH, D), jnp.float32),
        pltpu.VMEM_SHARED((NP, D), jnp.float32),
    ],
)
def _deg_kernel(dst_hbm, ones_hbm, zeros_hbm, degp_hbm, dstv, onesv, wbuf, acc):
    cid = lax.axis_index("c")
    sid = lax.axis_index("s")
    wid = cid * NS + sid
    pltpu.sync_copy(ones_hbm, onesv)
    pltpu.sync_copy(zeros_hbm, wbuf)
    row0 = sid * ROWS_PER_TILE
    for kk in range(WB):
        pltpu.sync_copy(wbuf, acc.at[pl.ds(row0 + kk * CH, CH)])
    plsc.subcore_barrier()

    def outer(r, _):
        pltpu.sync_copy(dst_hbm.at[wid, pl.ds(r * IB, IB)], dstv)

        def body(j, _):
            pltpu.sync_copy(onesv, acc.at[dstv.at[j]], add=True)
            return 0
        lax.fori_loop(0, IB, body, 0)
        return 0
    lax.fori_loop(0, NREFILL, outer, 0)
    plsc.subcore_barrier()

    def wb(kk, _):
        r = row0 + kk * CH
        pltpu.sync_copy(acc.at[pl.ds(r, CH)], wbuf)
        pltpu.sync_copy(wbuf, degp_hbm.at[cid, pl.ds(r, CH)])
        return 0
    lax.fori_loop(0, WB, wb, 0)


@functools.partial(
    pl.kernel,
    out_type=jax.ShapeDtypeStruct((NC, NP, D), jnp.float32),
    mesh=_mesh,
    scratch_types=[
        pltpu.VMEM((AIB, ACH), jnp.int32),
        pltpu.VMEM((AIB, ACH), jnp.int32),
        pltpu.VMEM((ACH, D), jnp.float32),
        pltpu.VMEM((ACH, D), jnp.float32),
        pltpu.VMEM((ACH, D), jnp.float32),
        pltpu.VMEM((ACH, D), jnp.float32),
        pltpu.VMEM_SHARED((NP, D), jnp.float32),
        pltpu.SemaphoreType.DMA,
        pltpu.SemaphoreType.DMA,
        pltpu.SemaphoreType.DMA,
        pltpu.SemaphoreType.DMA,
    ],
)
def _agg_kernel(hs_hbm, src_hbm, dst_hbm, zeros_hbm, part_hbm,
                srcv, dstv, b0, b1, b2, b3, acc, sem0, sem1, sem2, sem3):
    bufs = (b0, b1, b2, b3)
    sems = (sem0, sem1, sem2, sem3)
    cid = lax.axis_index("c")
    sid = lax.axis_index("s")
    wid = cid * NS + sid

    pltpu.sync_copy(zeros_hbm, b0)
    row0 = sid * ROWS_PER_TILE
    for kk in range(AWB):
        pltpu.sync_copy(b0, acc.at[pl.ds(row0 + kk * ACH, ACH)])
    plsc.subcore_barrier()

    def outer(r, _):
        pltpu.sync_copy(src_hbm.at[wid, pl.ds(r * AIB, AIB)], srcv)
        pltpu.sync_copy(dst_hbm.at[wid, pl.ds(r * AIB, AIB)], dstv)
        handles = [None] * AIB
        for j in range(AIB):
            b = j % NBUF
            if j >= NBUF:
                handles[j - NBUF].wait()
                pltpu.sync_copy(bufs[b], acc.at[dstv.at[j - NBUF]], add=True)
            handles[j] = pltpu.async_copy(hs_hbm.at[srcv.at[j]], bufs[b],
                                          sems[b])
        for j in range(AIB - NBUF, AIB):
            handles[j].wait()
            pltpu.sync_copy(bufs[j % NBUF], acc.at[dstv.at[j]], add=True)
        return 0
    lax.fori_loop(0, ANREFILL, outer, 0)
    plsc.subcore_barrier()

    def wb(kk, _):
        r = row0 + kk * ACH
        pltpu.sync_copy(acc.at[pl.ds(r, ACH)], b0)
        pltpu.sync_copy(b0, part_hbm.at[cid, pl.ds(r, ACH)])
        return 0
    lax.fori_loop(0, AWB, wb, 0)


BM = 1280
GRID = NP // BM


def _prep_body(degp_ref, x_ref, w_ref, hs_ref, dis_ref):
    deg = degp_ref[0, :, 0:1] + degp_ref[1, :, 0:1] + 1.0
    dis = jnp.broadcast_to(lax.rsqrt(deg), (BM, D))
    hs_ref[...] = jnp.dot(x_ref[...], w_ref[...],
                          preferred_element_type=jnp.float32) * dis
    dis_ref[...] = dis


_prep_call = pl.pallas_call(
    _prep_body,
    grid=(GRID,),
    in_specs=[
        pl.BlockSpec((NC, BM, D), lambda i: (0, i, 0)),
        pl.BlockSpec((BM, D), lambda i: (i, 0)),
        pl.BlockSpec((D, D), lambda i: (0, 0)),
    ],
    out_specs=[
        pl.BlockSpec((BM, D), lambda i: (i, 0)),
        pl.BlockSpec((BM, D), lambda i: (i, 0)),
    ],
    out_shape=[
        jax.ShapeDtypeStruct((NP, D), jnp.float32),
        jax.ShapeDtypeStruct((NP, D), jnp.float32),
    ],
)


def _mid_body(part_ref, hs_ref, dis_ref, b_ref, w_ref, out_ref):
    agg = part_ref[0] + part_ref[1] + hs_ref[...]
    o = dis_ref[...] * agg + b_ref[...]
    r = jnp.maximum(o, 0.0)
    out_ref[...] = jnp.dot(r, w_ref[...],
                           preferred_element_type=jnp.float32) * dis_ref[...]


_mid_call = pl.pallas_call(
    _mid_body,
    grid=(GRID,),
    in_specs=[
        pl.BlockSpec((NC, BM, D), lambda i: (0, i, 0)),
        pl.BlockSpec((BM, D), lambda i: (i, 0)),
        pl.BlockSpec((BM, D), lambda i: (i, 0)),
        pl.BlockSpec((1, D), lambda i: (0, 0)),
        pl.BlockSpec((D, D), lambda i: (0, 0)),
    ],
    out_specs=pl.BlockSpec((BM, D), lambda i: (i, 0)),
    out_shape=jax.ShapeDtypeStruct((NP, D), jnp.float32),
)


def _final_body(part_ref, hs_ref, dis_ref, b_ref, out_ref):
    agg = part_ref[0] + part_ref[1] + hs_ref[...]
    out_ref[...] = dis_ref[...] * agg + b_ref[...]


_final_call = pl.pallas_call(
    _final_body,
    grid=(GRID,),
    in_specs=[
        pl.BlockSpec((NC, BM, D), lambda i: (0, i, 0)),
        pl.BlockSpec((BM, D), lambda i: (i, 0)),
        pl.BlockSpec((BM, D), lambda i: (i, 0)),
        pl.BlockSpec((1, D), lambda i: (0, 0)),
    ],
    out_specs=pl.BlockSpec((BM, D), lambda i: (i, 0)),
    out_shape=jax.ShapeDtypeStruct((NP, D), jnp.float32),
)


def kernel(x, edge_index, W1, b1, W2, b2, W3, b3):
    src = edge_index[0].astype(jnp.int32)
    dst = edge_index[1].astype(jnp.int32)

    real_per_tile = N_EDGES // NW
    pad_per_tile = EPT - real_per_tile
    src_flat = jnp.concatenate(
        [src.reshape(NW, real_per_tile),
         jnp.zeros((NW, pad_per_tile), jnp.int32)], axis=1)
    trash = N_NODES + jnp.arange(pad_per_tile, dtype=jnp.int32)
    dst_flat = jnp.concatenate(
        [dst.reshape(NW, real_per_tile),
         jnp.broadcast_to(trash, (NW, pad_per_tile))], axis=1)
    src_t = src_flat.reshape(NW, ANCHUNK, ACH)
    dst_t = dst_flat.reshape(NW, ANCHUNK, ACH)
    dst_deg = dst_flat.reshape(NW, NCHUNK, CH)

    ones_hbm = jnp.ones((CH, D), jnp.float32)
    zeros_hbm = jnp.zeros((CH, D), jnp.float32)
    zeros_a = jnp.zeros((ACH, D), jnp.float32)
    xp = jnp.zeros((NP, D), jnp.float32).at[:N_NODES].set(x)
    b1r = b1.reshape(1, D)
    b2r = b2.reshape(1, D)
    b3r = b3.reshape(1, D)

    degp = _deg_kernel(dst_deg, ones_hbm, zeros_hbm)
    hs1, dis = _prep_call(degp, xp, W1)
    p1 = _agg_kernel(hs1, src_t, dst_t, zeros_a)
    hs2 = _mid_call(p1, hs1, dis, b1r, W2)
    p2 = _agg_kernel(hs2, src_t, dst_t, zeros_a)
    hs3 = _mid_call(p2, hs2, dis, b2r, W3)
    p3 = _agg_kernel(hs3, src_t, dst_t, zeros_a)
    z = _final_call(p3, hs3, dis, b3r)
    return z[:N_NODES]


# speedup vs baseline: 15.7845x; 1.5501x over previous
import functools

import jax
import jax.numpy as jnp
from jax import lax
from jax.experimental import pallas as pl
from jax.experimental.pallas import tpu as pltpu
from jax.experimental.pallas import tpu_sc as plsc

N_NODES = 10000
N_EDGES = 320000
D = 128

NC = 2
NS = 16
NW = NC * NS

NP = 10240
EPT = NP
CH = 128
NCHUNK = EPT // CH
IB = 16
NREFILL = NCHUNK // IB
ROWS_PER_TILE = NP // NS
WB = ROWS_PER_TILE // CH

ACH = 64
ANCHUNK = EPT // ACH
AIB = 32
ANREFILL = ANCHUNK // AIB
NBUF = 4
AWB = ROWS_PER_TILE // ACH

_mesh = plsc.VectorSubcoreMesh(core_axis_name="c", subcore_axis_name="s",
                               num_cores=NC, num_subcores=NS)


@functools.partial(
    pl.kernel,
    out_type=jax.ShapeDtypeStruct((NC, NP, D), jnp.float32),
    mesh=_mesh,
    scratch_types=[
        pltpu.VMEM((IB, CH), jnp.int32),
        pltpu.VMEM((CH, D), jnp.float32),
        pltpu.VMEM((CH, D), jnp.float32),
        pltpu.VMEM_SHARED((NP, D), jnp.float32),
    ],
)
def _deg_kernel(dst_hbm, ones_hbm, zeros_hbm, degp_hbm, dstv, onesv, wbuf, acc):
    cid = lax.axis_index("c")
    sid = lax.axis_index("s")
    wid = cid * NS + sid
    pltpu.sync_copy(ones_hbm, onesv)
    pltpu.sync_copy(zeros_hbm, wbuf)
    row0 = sid * ROWS_PER_TILE
    for kk in range(WB):
        pltpu.sync_copy(wbuf, acc.at[pl.ds(row0 + kk * CH, CH)])
    plsc.subcore_barrier()

    def outer(r, _):
        pltpu.sync_copy(dst_hbm.at[wid, pl.ds(r * IB, IB)], dstv)

        def body(j, _):
            pltpu.sync_copy(onesv, acc.at[dstv.at[j]], add=True)
            return 0
        lax.fori_loop(0, IB, body, 0)
        return 0
    lax.fori_loop(0, NREFILL, outer, 0)
    plsc.subcore_barrier()

    def wb(kk, _):
        r = row0 + kk * CH
        pltpu.sync_copy(acc.at[pl.ds(r, CH)], wbuf)
        pltpu.sync_copy(wbuf, degp_hbm.at[cid, pl.ds(r, CH)])
        return 0
    lax.fori_loop(0, WB, wb, 0)


@functools.partial(
    pl.kernel,
    out_type=jax.ShapeDtypeStruct((NC, NP, D), jnp.float32),
    mesh=_mesh,
    scratch_types=[
        pltpu.VMEM((AIB, ACH), jnp.int32),
        pltpu.VMEM((AIB, ACH), jnp.int32),
        pltpu.VMEM((ACH, D), jnp.float32),
        pltpu.VMEM((ACH, D), jnp.float32),
        pltpu.VMEM((ACH, D), jnp.float32),
        pltpu.VMEM((ACH, D), jnp.float32),
        pltpu.VMEM_SHARED((NP, D), jnp.float32),
        pltpu.SemaphoreType.DMA,
        pltpu.SemaphoreType.DMA,
        pltpu.SemaphoreType.DMA,
        pltpu.SemaphoreType.DMA,
    ],
)
def _agg_kernel(hs_hbm, src_hbm, dst_hbm, zeros_hbm, part_hbm,
                srcv, dstv, b0, b1, b2, b3, acc, sem0, sem1, sem2, sem3):
    bufs = (b0, b1, b2, b3)
    sems = (sem0, sem1, sem2, sem3)
    cid = lax.axis_index("c")
    sid = lax.axis_index("s")
    wid = cid * NS + sid

    pltpu.sync_copy(zeros_hbm, b0)
    row0 = sid * ROWS_PER_TILE
    for kk in range(AWB):
        pltpu.sync_copy(b0, acc.at[pl.ds(row0 + kk * ACH, ACH)])
    plsc.subcore_barrier()

    def outer(r, _):
        pltpu.sync_copy(src_hbm.at[wid, pl.ds(r * AIB, AIB)], srcv)
        pltpu.sync_copy(dst_hbm.at[wid, pl.ds(r * AIB, AIB)], dstv)
        handles = [None] * AIB
        for j in range(AIB):
            b = j % NBUF
            if j >= NBUF:
                handles[j - NBUF].wait()
                pltpu.sync_copy(bufs[b], acc.at[dstv.at[j - NBUF]], add=True)
            handles[j] = pltpu.async_copy(hs_hbm.at[srcv.at[j]], bufs[b],
                                          sems[b])
        for j in range(AIB - NBUF, AIB):
            handles[j].wait()
            pltpu.sync_copy(bufs[j % NBUF], acc.at[dstv.at[j]], add=True)
        return 0
    lax.fori_loop(0, ANREFILL, outer, 0)
    plsc.subcore_barrier()

    def wb(kk, _):
        r = row0 + kk * ACH
        pltpu.sync_copy(acc.at[pl.ds(r, ACH)], b0)
        pltpu.sync_copy(b0, part_hbm.at[cid, pl.ds(r, ACH)])
        return 0
    lax.fori_loop(0, AWB, wb, 0)


BM = 1280
GRID = NP // BM


def _prep_body(degp_ref, x_ref, w_ref, hs_ref, dis_ref):
    deg = degp_ref[0, :, 0:1] + degp_ref[1, :, 0:1] + 1.0
    dis = jnp.broadcast_to(lax.rsqrt(deg), (BM, D))
    hs_ref[...] = jnp.dot(x_ref[...], w_ref[...],
                          preferred_element_type=jnp.float32) * dis
    dis_ref[...] = dis


_prep_call = pl.pallas_call(
    _prep_body,
    grid=(GRID,),
    in_specs=[
        pl.BlockSpec((NC, BM, D), lambda i: (0, i, 0)),
        pl.BlockSpec((BM, D), lambda i: (i, 0)),
        pl.BlockSpec((D, D), lambda i: (0, 0)),
    ],
    out_specs=[
        pl.BlockSpec((BM, D), lambda i: (i, 0)),
        pl.BlockSpec((BM, D), lambda i: (i, 0)),
    ],
    out_shape=[
        jax.ShapeDtypeStruct((NP, D), jnp.float32),
        jax.ShapeDtypeStruct((NP, D), jnp.float32),
    ],
)


def _mid_body(part_ref, hs_ref, dis_ref, b_ref, w_ref, out_ref):
    agg = part_ref[0] + part_ref[1] + hs_ref[...]
    o = dis_ref[...] * agg + b_ref[...]
    r = jnp.maximum(o, 0.0)
    out_ref[...] = jnp.dot(r, w_ref[...],
                           preferred_element_type=jnp.float32) * dis_ref[...]


_mid_call = pl.pallas_call(
    _mid_body,
    grid=(GRID,),
    in_specs=[
        pl.BlockSpec((NC, BM, D), lambda i: (0, i, 0)),
        pl.BlockSpec((BM, D), lambda i: (i, 0)),
        pl.BlockSpec((BM, D), lambda i: (i, 0)),
        pl.BlockSpec((1, D), lambda i: (0, 0)),
        pl.BlockSpec((D, D), lambda i: (0, 0)),
    ],
    out_specs=pl.BlockSpec((BM, D), lambda i: (i, 0)),
    out_shape=jax.ShapeDtypeStruct((NP, D), jnp.float32),
)


def _final_body(part_ref, hs_ref, dis_ref, b_ref, out_ref):
    agg = part_ref[0] + part_ref[1] + hs_ref[...]
    out_ref[...] = dis_ref[...] * agg + b_ref[...]


_final_call = pl.pallas_call(
    _final_body,
    grid=(GRID,),
    in_specs=[
        pl.BlockSpec((NC, BM, D), lambda i: (0, i, 0)),
        pl.BlockSpec((BM, D), lambda i: (i, 0)),
        pl.BlockSpec((BM, D), lambda i: (i, 0)),
        pl.BlockSpec((1, D), lambda i: (0, 0)),
    ],
    out_specs=pl.BlockSpec((BM, D), lambda i: (i, 0)),
    out_shape=jax.ShapeDtypeStruct((NP, D), jnp.float32),
)


def kernel(x, edge_index, W1, b1, W2, b2, W3, b3):
    src = edge_index[0].astype(jnp.int32)
    dst = edge_index[1].astype(jnp.int32)

    real_per_tile = N_EDGES // NW
    pad_per_tile = EPT - real_per_tile
    pad_src = N_NODES + jnp.arange(pad_per_tile, dtype=jnp.int32)
    src_flat = jnp.concatenate(
        [src.reshape(NW, real_per_tile),
         jnp.broadcast_to(pad_src, (NW, pad_per_tile))], axis=1)
    trash = N_NODES + jnp.arange(pad_per_tile, dtype=jnp.int32)
    dst_flat = jnp.concatenate(
        [dst.reshape(NW, real_per_tile),
         jnp.broadcast_to(trash, (NW, pad_per_tile))], axis=1)
    order = jnp.argsort(src_flat, axis=1)
    src_flat = jnp.take_along_axis(src_flat, order, axis=1)
    dst_flat = jnp.take_along_axis(dst_flat, order, axis=1)
    src_t = src_flat.reshape(NW, ANCHUNK, ACH)
    dst_t = dst_flat.reshape(NW, ANCHUNK, ACH)
    dst_deg = dst_flat.reshape(NW, NCHUNK, CH)

    ones_hbm = jnp.ones((CH, D), jnp.float32)
    zeros_hbm = jnp.zeros((CH, D), jnp.float32)
    zeros_a = jnp.zeros((ACH, D), jnp.float32)
    xp = jnp.zeros((NP, D), jnp.float32).at[:N_NODES].set(x)
    b1r = b1.reshape(1, D)
    b2r = b2.reshape(1, D)
    b3r = b3.reshape(1, D)

    degp = _deg_kernel(dst_deg, ones_hbm, zeros_hbm)
    hs1, dis = _prep_call(degp, xp, W1)
    p1 = _agg_kernel(hs1, src_t, dst_t, zeros_a)
    hs2 = _mid_call(p1, hs1, dis, b1r, W2)
    p2 = _agg_kernel(hs2, src_t, dst_t, zeros_a)
    hs3 = _mid_call(p2, hs2, dis, b2r, W3)
    p3 = _agg_kernel(hs3, src_t, dst_t, zeros_a)
    z = _final_call(p3, hs3, dis, b3r)
    return z[:N_NODES]


# speedup vs baseline: 18.3715x; 1.1639x over previous
import functools

import jax
import jax.numpy as jnp
from jax import lax
from jax.experimental import pallas as pl
from jax.experimental.pallas import tpu as pltpu
from jax.experimental.pallas import tpu_sc as plsc

N_NODES = 10000
N_EDGES = 320000
D = 128

NC = 2
NS = 16
NW = NC * NS

NP = 10240
EPT = NP
CH = 128
NCHUNK = EPT // CH
IB = 16
NREFILL = NCHUNK // IB
ROWS_PER_TILE = NP // NS
WB = ROWS_PER_TILE // CH

ACH = 64
ANCHUNK = EPT // ACH
AIB = 32
ANREFILL = ANCHUNK // AIB
NBUF = 4
AWB = ROWS_PER_TILE // ACH

_mesh = plsc.VectorSubcoreMesh(core_axis_name="c", subcore_axis_name="s",
                               num_cores=NC, num_subcores=NS)


@functools.partial(
    pl.kernel,
    out_type=jax.ShapeDtypeStruct((NC, NP, D), jnp.float32),
    mesh=_mesh,
    scratch_types=[
        pltpu.VMEM((IB, CH), jnp.int32),
        pltpu.VMEM((CH, D), jnp.float32),
        pltpu.VMEM((CH, D), jnp.float32),
        pltpu.VMEM_SHARED((NP, D), jnp.float32),
    ],
)
def _deg_kernel(dst_hbm, ones_hbm, zeros_hbm, degp_hbm, dstv, onesv, wbuf, acc):
    cid = lax.axis_index("c")
    sid = lax.axis_index("s")
    wid = cid * NS + sid
    pltpu.sync_copy(ones_hbm, onesv)
    pltpu.sync_copy(zeros_hbm, wbuf)
    row0 = sid * ROWS_PER_TILE
    for kk in range(WB):
        pltpu.sync_copy(wbuf, acc.at[pl.ds(row0 + kk * CH, CH)])
    plsc.subcore_barrier()

    def outer(r, _):
        pltpu.sync_copy(dst_hbm.at[wid, pl.ds(r * IB, IB)], dstv)

        def body(j, _):
            pltpu.sync_copy(onesv, acc.at[dstv.at[j]], add=True)
            return 0
        lax.fori_loop(0, IB, body, 0)
        return 0
    lax.fori_loop(0, NREFILL, outer, 0)
    plsc.subcore_barrier()

    def wb(kk, _):
        r = row0 + kk * CH
        pltpu.sync_copy(acc.at[pl.ds(r, CH)], wbuf)
        pltpu.sync_copy(wbuf, degp_hbm.at[cid, pl.ds(r, CH)])
        return 0
    lax.fori_loop(0, WB, wb, 0)


@functools.partial(
    pl.kernel,
    out_type=jax.ShapeDtypeStruct((NC, NP, D), jnp.float32),
    mesh=_mesh,
    scratch_types=[
        pltpu.VMEM((AIB, ACH), jnp.int32),
        pltpu.VMEM((AIB, ACH), jnp.int32),
        pltpu.VMEM((ACH, D), jnp.float32),
        pltpu.VMEM((ACH, D), jnp.float32),
        pltpu.VMEM((ACH, D), jnp.float32),
        pltpu.VMEM((ACH, D), jnp.float32),
        pltpu.VMEM_SHARED((NP, D), jnp.float32),
        pltpu.SemaphoreType.DMA,
        pltpu.SemaphoreType.DMA,
        pltpu.SemaphoreType.DMA,
        pltpu.SemaphoreType.DMA,
    ],
)
def _agg_kernel(hs_hbm, src_hbm, dst_hbm, zeros_hbm, part_hbm,
                srcv, dstv, b0, b1, b2, b3, acc, sem0, sem1, sem2, sem3):
    bufs = (b0, b1, b2, b3)
    sems = (sem0, sem1, sem2, sem3)
    cid = lax.axis_index("c")
    sid = lax.axis_index("s")
    wid = cid * NS + sid

    pltpu.sync_copy(zeros_hbm, b0)
    row0 = sid * ROWS_PER_TILE
    for kk in range(AWB):
        pltpu.sync_copy(b0, acc.at[pl.ds(row0 + kk * ACH, ACH)])
    plsc.subcore_barrier()

    def outer(r, _):
        pltpu.sync_copy(src_hbm.at[wid, pl.ds(r * AIB, AIB)], srcv)
        pltpu.sync_copy(dst_hbm.at[wid, pl.ds(r * AIB, AIB)], dstv)
        handles = [None] * AIB
        for j in range(AIB):
            b = j % NBUF
            if j >= NBUF:
                handles[j - NBUF].wait()
                pltpu.sync_copy(bufs[b], acc.at[dstv.at[j - NBUF]], add=True)
            handles[j] = pltpu.async_copy(hs_hbm.at[srcv.at[j]], bufs[b],
                                          sems[b])
        for j in range(AIB - NBUF, AIB):
            handles[j].wait()
            pltpu.sync_copy(bufs[j % NBUF], acc.at[dstv.at[j]], add=True)
        return 0
    lax.fori_loop(0, ANREFILL, outer, 0)
    plsc.subcore_barrier()

    def wb(kk, _):
        r = row0 + kk * ACH
        pltpu.sync_copy(acc.at[pl.ds(r, ACH)], b0)
        pltpu.sync_copy(b0, part_hbm.at[cid, pl.ds(r, ACH)])
        return 0
    lax.fori_loop(0, AWB, wb, 0)


BM = 1280
GRID = NP // BM


def _prep_body(degp_ref, x_ref, w_ref, hs_ref, dis_ref):
    deg = degp_ref[0, :, 0:1] + degp_ref[1, :, 0:1] + 1.0
    dis = jnp.broadcast_to(lax.rsqrt(deg), (BM, D))
    hs_ref[...] = jnp.dot(x_ref[...], w_ref[...],
                          preferred_element_type=jnp.float32) * dis
    dis_ref[...] = dis


_prep_call = pl.pallas_call(
    _prep_body,
    grid=(GRID,),
    in_specs=[
        pl.BlockSpec((NC, BM, D), lambda i: (0, i, 0)),
        pl.BlockSpec((BM, D), lambda i: (i, 0)),
        pl.BlockSpec((D, D), lambda i: (0, 0)),
    ],
    out_specs=[
        pl.BlockSpec((BM, D), lambda i: (i, 0)),
        pl.BlockSpec((BM, D), lambda i: (i, 0)),
    ],
    out_shape=[
        jax.ShapeDtypeStruct((NP, D), jnp.float32),
        jax.ShapeDtypeStruct((NP, D), jnp.float32),
    ],
)


def _mid_body(part_ref, hs_ref, dis_ref, b_ref, w_ref, out_ref):
    agg = part_ref[0] + part_ref[1] + hs_ref[...]
    o = dis_ref[...] * agg + b_ref[...]
    r = jnp.maximum(o, 0.0)
    out_ref[...] = jnp.dot(r, w_ref[...],
                           preferred_element_type=jnp.float32) * dis_ref[...]


_mid_call = pl.pallas_call(
    _mid_body,
    grid=(GRID,),
    in_specs=[
        pl.BlockSpec((NC, BM, D), lambda i: (0, i, 0)),
        pl.BlockSpec((BM, D), lambda i: (i, 0)),
        pl.BlockSpec((BM, D), lambda i: (i, 0)),
        pl.BlockSpec((1, D), lambda i: (0, 0)),
        pl.BlockSpec((D, D), lambda i: (0, 0)),
    ],
    out_specs=pl.BlockSpec((BM, D), lambda i: (i, 0)),
    out_shape=jax.ShapeDtypeStruct((NP, D), jnp.float32),
)


def _final_body(part_ref, hs_ref, dis_ref, b_ref, out_ref):
    agg = part_ref[0] + part_ref[1] + hs_ref[...]
    out_ref[...] = dis_ref[...] * agg + b_ref[...]


_final_call = pl.pallas_call(
    _final_body,
    grid=(GRID,),
    in_specs=[
        pl.BlockSpec((NC, BM, D), lambda i: (0, i, 0)),
        pl.BlockSpec((BM, D), lambda i: (i, 0)),
        pl.BlockSpec((BM, D), lambda i: (i, 0)),
        pl.BlockSpec((1, D), lambda i: (0, 0)),
    ],
    out_specs=pl.BlockSpec((BM, D), lambda i: (i, 0)),
    out_shape=jax.ShapeDtypeStruct((NP, D), jnp.float32),
)


def kernel(x, edge_index, W1, b1, W2, b2, W3, b3):
    src = edge_index[0].astype(jnp.int32)
    dst = edge_index[1].astype(jnp.int32)

    real_per_tile = N_EDGES // NW
    pad_per_tile = EPT - real_per_tile
    pad_src = N_NODES + jnp.arange(pad_per_tile, dtype=jnp.int32)
    src_flat = jnp.concatenate(
        [src.reshape(NW, real_per_tile),
         jnp.broadcast_to(pad_src, (NW, pad_per_tile))], axis=1)
    trash = N_NODES + jnp.arange(pad_per_tile, dtype=jnp.int32)
    dst_flat = jnp.concatenate(
        [dst.reshape(NW, real_per_tile),
         jnp.broadcast_to(trash, (NW, pad_per_tile))], axis=1)
    dst_deg = dst_flat.reshape(NW, NCHUNK, CH)
    src_flat, dst_flat = lax.sort_key_val(src_flat, dst_flat, dimension=1)
    src_t = src_flat.reshape(NW, ANCHUNK, ACH)
    dst_t = dst_flat.reshape(NW, ANCHUNK, ACH)

    ones_hbm = jnp.ones((CH, D), jnp.float32)
    zeros_hbm = jnp.zeros((CH, D), jnp.float32)
    zeros_a = jnp.zeros((ACH, D), jnp.float32)
    xp = jnp.zeros((NP, D), jnp.float32).at[:N_NODES].set(x)
    b1r = b1.reshape(1, D)
    b2r = b2.reshape(1, D)
    b3r = b3.reshape(1, D)

    degp = _deg_kernel(dst_deg, ones_hbm, zeros_hbm)
    hs1, dis = _prep_call(degp, xp, W1)
    p1 = _agg_kernel(hs1, src_t, dst_t, zeros_a)
    hs2 = _mid_call(p1, hs1, dis, b1r, W2)
    p2 = _agg_kernel(hs2, src_t, dst_t, zeros_a)
    hs3 = _mid_call(p2, hs2, dis, b2r, W3)
    p3 = _agg_kernel(hs3, src_t, dst_t, zeros_a)
    z = _final_call(p3, hs3, dis, b3r)
    return z[:N_NODES]
